# Initial kernel scaffold; baseline (speedup 1.0000x reference)
import jax
import jax.numpy as jnp
from jax.experimental import pallas as pl


def kernel(inputs, database, weights, equity_weight):
    raise NotImplementedError("write your pallas kernel here")


# speedup vs baseline: 23.3501x; 23.3501x over previous
import functools

import jax
import jax.numpy as jnp
from jax.experimental import pallas as pl

BATCH = 8
WIDTH = 32
N_NODE = 2048
N_REL = 4
JB = 512
NJ = N_NODE // JB


def _lerp_kernel(db_ref, x_ref, w1_ref, w2_ref, eq0_ref, eq1_ref, out_ref):
    r = pl.program_id(0)
    j = pl.program_id(1)
    step = r * NJ + j
    nsteps = N_REL * NJ

    @pl.when(step == 0)
    def _init():
        out_ref[...] = jnp.zeros_like(out_ref)

    tile = db_ref[0]
    x = x_ref[...]
    w1r = w1_ref[0]
    w2r = w2_ref[0]

    xs1 = x_ref[:, pl.ds(j * JB, JB)] * w1r
    y1 = jax.lax.dot_general(
        xs1, tile, (((1,), (0,)), ((), ())), preferred_element_type=jnp.float32
    )
    out_ref[...] += y1

    xs2 = x * w2r
    y2 = jax.lax.dot_general(
        xs2, tile, (((1,), (1,)), ((), ())), preferred_element_type=jnp.float32
    )
    out_ref[:, pl.ds(j * JB, JB)] += y2

    @pl.when(step == nsteps - 1)
    def _fin():
        acc = out_ref[...]
        eq0 = eq0_ref[...]
        eq1 = eq1_ref[...]
        out_ref[...] = (1.0 - jnp.exp(-acc)) * eq0 + x * eq1


@jax.jit
def kernel(inputs, database, weights, equity_weight):
    m = BATCH * WIDTH
    x = inputs.reshape(m, N_NODE)
    w = jax.nn.softmax(weights, axis=1)
    w1 = jnp.tile(w[:, :N_REL].T[:, None, :], (1, BATCH, 1)).reshape(N_REL, m, 1)
    w2 = jnp.tile(w[:, N_REL:].T[:, None, :], (1, BATCH, 1)).reshape(N_REL, m, 1)
    eq = jnp.tile(jax.nn.softmax(equity_weight, axis=1), (BATCH, 1))
    eq0 = eq[:, 0:1]
    eq1 = eq[:, 1:2]

    out2d = pl.pallas_call(
        _lerp_kernel,
        grid=(N_REL, NJ),
        in_specs=[
            pl.BlockSpec((1, JB, N_NODE), lambda r, j: (r, j, 0)),
            pl.BlockSpec((m, N_NODE), lambda r, j: (0, 0)),
            pl.BlockSpec((1, m, 1), lambda r, j: (r, 0, 0)),
            pl.BlockSpec((1, m, 1), lambda r, j: (r, 0, 0)),
            pl.BlockSpec((m, 1), lambda r, j: (0, 0)),
            pl.BlockSpec((m, 1), lambda r, j: (0, 0)),
        ],
        out_specs=pl.BlockSpec((m, N_NODE), lambda r, j: (0, 0)),
        out_shape=jax.ShapeDtypeStruct((m, N_NODE), jnp.float32),
    )(database, x, w1, w2, eq0, eq1)
    return out2d.reshape(BATCH, WIDTH, N_NODE)


# speedup vs baseline: 23.4196x; 1.0030x over previous
import functools

import jax
import jax.numpy as jnp
from jax.experimental import pallas as pl

BATCH = 8
WIDTH = 32
N_NODE = 2048
N_REL = 4
JB = 512
NJ = N_NODE // JB


def _lerp_kernel(db_ref, x_ref, w1_ref, w2_ref, eq0_ref, eq1_ref, out_ref):
    r = pl.program_id(0)
    j = pl.program_id(1)
    step = r * NJ + j
    nsteps = N_REL * NJ

    @pl.when(step == 0)
    def _init():
        out_ref[...] = jnp.zeros_like(out_ref)

    tile = db_ref[0]
    x = x_ref[...]
    w1r = w1_ref[0]
    w2r = w2_ref[0]

    tile_bf = tile.astype(jnp.bfloat16)

    xs1 = (x_ref[:, pl.ds(j * JB, JB)] * w1r).astype(jnp.bfloat16)
    y1 = jax.lax.dot_general(
        xs1, tile_bf, (((1,), (0,)), ((), ())), preferred_element_type=jnp.float32
    )
    out_ref[...] += y1

    xs2 = (x * w2r).astype(jnp.bfloat16)
    y2 = jax.lax.dot_general(
        xs2, tile_bf, (((1,), (1,)), ((), ())), preferred_element_type=jnp.float32
    )
    out_ref[:, pl.ds(j * JB, JB)] += y2

    @pl.when(step == nsteps - 1)
    def _fin():
        acc = out_ref[...]
        eq0 = eq0_ref[...]
        eq1 = eq1_ref[...]
        out_ref[...] = (1.0 - jnp.exp(-acc)) * eq0 + x * eq1


@jax.jit
def kernel(inputs, database, weights, equity_weight):
    m = BATCH * WIDTH
    x = inputs.reshape(m, N_NODE)
    w = jax.nn.softmax(weights, axis=1)
    w1 = jnp.tile(w[:, :N_REL].T[:, None, :], (1, BATCH, 1)).reshape(N_REL, m, 1)
    w2 = jnp.tile(w[:, N_REL:].T[:, None, :], (1, BATCH, 1)).reshape(N_REL, m, 1)
    eq = jnp.tile(jax.nn.softmax(equity_weight, axis=1), (BATCH, 1))
    eq0 = eq[:, 0:1]
    eq1 = eq[:, 1:2]

    out2d = pl.pallas_call(
        _lerp_kernel,
        grid=(N_REL, NJ),
        in_specs=[
            pl.BlockSpec((1, JB, N_NODE), lambda r, j: (r, j, 0)),
            pl.BlockSpec((m, N_NODE), lambda r, j: (0, 0)),
            pl.BlockSpec((1, m, 1), lambda r, j: (r, 0, 0)),
            pl.BlockSpec((1, m, 1), lambda r, j: (r, 0, 0)),
            pl.BlockSpec((m, 1), lambda r, j: (0, 0)),
            pl.BlockSpec((m, 1), lambda r, j: (0, 0)),
        ],
        out_specs=pl.BlockSpec((m, N_NODE), lambda r, j: (0, 0)),
        out_shape=jax.ShapeDtypeStruct((m, N_NODE), jnp.float32),
    )(database, x, w1, w2, eq0, eq1)
    return out2d.reshape(BATCH, WIDTH, N_NODE)


# speedup vs baseline: 23.6421x; 1.0095x over previous
import jax
import jax.numpy as jnp
from jax.experimental import pallas as pl

BATCH = 8
WIDTH = 32
N_NODE = 2048
N_REL = 4


def _lerp_kernel(db_ref, x_ref, w1_ref, w2_ref, eq0_ref, eq1_ref, out_ref):
    r = pl.program_id(0)

    d = db_ref[0].astype(jnp.bfloat16)
    x = x_ref[...]
    xs1 = (x * w1_ref[0]).astype(jnp.bfloat16)
    xs2 = (x * w2_ref[0]).astype(jnp.bfloat16)

    y = jax.lax.dot_general(
        xs1, d, (((1,), (0,)), ((), ())), preferred_element_type=jnp.float32
    )
    y += jax.lax.dot_general(
        xs2, d, (((1,), (1,)), ((), ())), preferred_element_type=jnp.float32
    )

    @pl.when(r == 0)
    def _first():
        out_ref[...] = y

    @pl.when(r > 0)
    def _rest():
        out_ref[...] += y

    @pl.when(r == N_REL - 1)
    def _fin():
        acc = out_ref[...]
        out_ref[...] = (1.0 - jnp.exp(-acc)) * eq0_ref[...] + x * eq1_ref[...]


@jax.jit
def kernel(inputs, database, weights, equity_weight):
    m = BATCH * WIDTH
    x = inputs.reshape(m, N_NODE)
    w = jax.nn.softmax(weights, axis=1)
    w1 = jnp.tile(w[:, :N_REL].T[:, None, :], (1, BATCH, 1)).reshape(N_REL, m, 1)
    w2 = jnp.tile(w[:, N_REL:].T[:, None, :], (1, BATCH, 1)).reshape(N_REL, m, 1)
    eq = jnp.tile(jax.nn.softmax(equity_weight, axis=1), (BATCH, 1))
    eq0 = eq[:, 0:1]
    eq1 = eq[:, 1:2]

    out2d = pl.pallas_call(
        _lerp_kernel,
        grid=(N_REL,),
        in_specs=[
            pl.BlockSpec((1, N_NODE, N_NODE), lambda r: (r, 0, 0)),
            pl.BlockSpec((m, N_NODE), lambda r: (0, 0)),
            pl.BlockSpec((1, m, 1), lambda r: (r, 0, 0)),
            pl.BlockSpec((1, m, 1), lambda r: (r, 0, 0)),
            pl.BlockSpec((m, 1), lambda r: (0, 0)),
            pl.BlockSpec((m, 1), lambda r: (0, 0)),
        ],
        out_specs=pl.BlockSpec((m, N_NODE), lambda r: (0, 0)),
        out_shape=jax.ShapeDtypeStruct((m, N_NODE), jnp.float32),
    )(database, x, w1, w2, eq0, eq1)
    return out2d.reshape(BATCH, WIDTH, N_NODE)


# speedup vs baseline: 29.5808x; 1.2512x over previous
import jax
import jax.numpy as jnp
from jax.experimental import pallas as pl

BATCH = 8
WIDTH = 32
N_NODE = 2048
N_REL = 4


def _rowscale(col):
    return jnp.concatenate([col] * BATCH, axis=0)


def _lerp_kernel(db_ref, x_ref, w_ref, eq_ref, out_ref):
    r = pl.program_id(0)

    wsm = jax.nn.softmax(w_ref[...], axis=1)

    def sel(base):
        c = wsm[:, base + N_REL - 1 : base + N_REL]
        for k in range(N_REL - 2, -1, -1):
            c = jnp.where(r == k, wsm[:, base + k : base + k + 1], c)
        return c

    w1m = _rowscale(sel(0))
    w2m = _rowscale(sel(N_REL))

    d = db_ref[0].astype(jnp.bfloat16)
    x = x_ref[...]
    xs1 = (x * w1m).astype(jnp.bfloat16)
    xs2 = (x * w2m).astype(jnp.bfloat16)

    y = jax.lax.dot_general(
        xs1, d, (((1,), (0,)), ((), ())), preferred_element_type=jnp.float32
    )
    y += jax.lax.dot_general(
        xs2, d, (((1,), (1,)), ((), ())), preferred_element_type=jnp.float32
    )

    @pl.when(r == 0)
    def _first():
        out_ref[...] = y

    @pl.when(r > 0)
    def _rest():
        out_ref[...] += y

    @pl.when(r == N_REL - 1)
    def _fin():
        eqsm = jax.nn.softmax(eq_ref[...], axis=1)
        eq0 = _rowscale(eqsm[:, 0:1])
        eq1 = _rowscale(eqsm[:, 1:2])
        acc = out_ref[...]
        out_ref[...] = (1.0 - jnp.exp(-acc)) * eq0 + x * eq1


@jax.jit
def kernel(inputs, database, weights, equity_weight):
    m = BATCH * WIDTH
    x = inputs.reshape(m, N_NODE)
    out2d = pl.pallas_call(
        _lerp_kernel,
        grid=(N_REL,),
        in_specs=[
            pl.BlockSpec((1, N_NODE, N_NODE), lambda r: (r, 0, 0)),
            pl.BlockSpec((m, N_NODE), lambda r: (0, 0)),
            pl.BlockSpec((WIDTH, 2 * N_REL), lambda r: (0, 0)),
            pl.BlockSpec((WIDTH, 2), lambda r: (0, 0)),
        ],
        out_specs=pl.BlockSpec((m, N_NODE), lambda r: (0, 0)),
        out_shape=jax.ShapeDtypeStruct((m, N_NODE), jnp.float32),
    )(database, x, weights, equity_weight)
    return out2d.reshape(BATCH, WIDTH, N_NODE)
